# baseline (speedup 1.0000x reference)
import functools

import jax
import jax.numpy as jnp
from jax import lax
from jax.experimental import pallas as pl
from jax.experimental.pallas import tpu as pltpu
from jax.experimental.pallas import tpu_sc as plsc

N = 10000
NS = N + 16
E = 320000
BE = 128
NBLK = 2560
E_PAD = NBLK * BE
BN_EPS = 1e-5
BM = 1000

_MESH = plsc.VectorSubcoreMesh(core_axis_name="c", subcore_axis_name="s")
_SC_PARAMS = pltpu.CompilerParams(use_tc_tiling_on_sc=False)


@functools.partial(
    pl.kernel,
    out_type=jax.ShapeDtypeStruct((2, N, 16), jnp.float32),
    mesh=_MESH,
    scratch_types=[
        pltpu.VMEM((NBLK // 32, BE), jnp.int32),
        pltpu.VMEM((BE, 16), jnp.float32),
        pltpu.VMEM((125, 16), jnp.float32),
        pltpu.VMEM_SHARED((NS, 16), jnp.float32),
    ],
    compiler_params=_SC_PARAMS,
)
def _deg_kernel(dst_hbm, ones_hbm, zeros_hbm, out_hbm,
                dst_v, ones_v, obuf, acc_sh):
    cid = lax.axis_index("c")
    sid = lax.axis_index("s")
    nbw = NBLK // 32
    pltpu.sync_copy(zeros_hbm.at[pl.ds(sid * 626, 626)],
                    acc_sh.at[pl.ds(sid * 626, 626)])
    pltpu.sync_copy(ones_hbm, ones_v)
    base = cid * (NBLK // 2) + sid * nbw
    pltpu.sync_copy(dst_hbm.at[pl.ds(base, nbw)], dst_v)
    plsc.subcore_barrier()

    def body(b, carry):
        pltpu.sync_copy(ones_v, acc_sh.at[dst_v.at[b]], add=True)
        return carry

    lax.fori_loop(0, nbw, body, 0)
    plsc.subcore_barrier()
    for j in range(5):
        r = sid * 625 + j * 125
        pltpu.sync_copy(acc_sh.at[pl.ds(r, 125)], obuf)
        pltpu.sync_copy(obuf, out_hbm.at[cid, pl.ds(r, 125)])


_NBW = NBLK // 32
_K = 4


@functools.partial(
    pl.kernel,
    out_type=jax.ShapeDtypeStruct((2, N, 128), jnp.bfloat16),
    mesh=_MESH,
    scratch_types=[
        pltpu.VMEM((_NBW, BE), jnp.int32),
        pltpu.VMEM((_NBW, BE), jnp.int32),
        [pltpu.VMEM((BE, 128), jnp.bfloat16)] * _K,
        pltpu.VMEM((125, 128), jnp.bfloat16),
        pltpu.VMEM_SHARED((NS, 128), jnp.bfloat16),
        [pltpu.SemaphoreType.DMA] * _K,
    ],
    compiler_params=_SC_PARAMS,
)
def _agg(y_hbm, src_hbm, dst_hbm, zeros_hbm, out_hbm,
         src_v, dst_v, rows, obuf, acc_sh, sems):
    cid = lax.axis_index("c")
    sid = lax.axis_index("s")
    pltpu.sync_copy(zeros_hbm.at[pl.ds(sid * 626, 626)],
                    acc_sh.at[pl.ds(sid * 626, 626)])
    base = cid * (NBLK // 2) + sid * _NBW
    pltpu.sync_copy(src_hbm.at[pl.ds(base, _NBW)], src_v)
    pltpu.sync_copy(dst_hbm.at[pl.ds(base, _NBW)], dst_v)
    plsc.subcore_barrier()

    def body(g, carry):
        b = g * _K
        for k in range(_K):
            pltpu.async_copy(y_hbm.at[src_v.at[b + k]], rows[k], sems[k])
        for k in range(_K):
            pltpu.make_async_copy(y_hbm.at[src_v.at[b + k]], rows[k],
                                  sems[k]).wait()
            pltpu.sync_copy(rows[k], acc_sh.at[dst_v.at[b + k]], add=True)
        return carry

    lax.fori_loop(0, _NBW // _K, body, 0)
    plsc.subcore_barrier()
    for j in range(5):
        r = sid * 625 + j * 125
        pltpu.sync_copy(acc_sh.at[pl.ds(r, 125)], obuf)
        pltpu.sync_copy(obuf, out_hbm.at[cid, pl.ds(r, 125)])


def _dis_block(parts):
    deg = parts[0][:, 0:1] + parts[1][:, 0:1] + 1.0
    return lax.rsqrt(deg)


def _psum(ref):
    return (ref[0] + ref[1]).astype(jnp.float32)


def _tc0_body(x_ref, w_ref, parts_ref, out_ref):
    dis = _dis_block(parts_ref)
    y = (jnp.dot(x_ref[...], w_ref[...],
                 preferred_element_type=jnp.float32) * dis).astype(
                     jnp.bfloat16)
    out_ref[0] = y[:, :128]
    out_ref[1] = y[:, 128:]


def _tc1_body(agga_ref, aggb_ref, y_ref, parts_ref, b_ref, g_ref, be_ref,
              rm_ref, rv_ref, w_ref, out_ref):
    dis = _dis_block(parts_ref)
    agg = jnp.concatenate([_psum(agga_ref), _psum(aggb_ref)], axis=1)
    ysl = jnp.concatenate([y_ref[0], y_ref[1]], axis=1).astype(jnp.float32)
    h = (agg + ysl) * dis + b_ref[...]
    h = jnp.maximum(h, 0.0)
    scale = g_ref[...] * lax.rsqrt(rv_ref[...] + BN_EPS)
    h = (h - rm_ref[...]) * scale + be_ref[...]
    out_ref[...] = jnp.dot(h * dis, w_ref[...],
                           preferred_element_type=jnp.float32).astype(
                               jnp.bfloat16)


def _tc2_body(agg_ref, y_ref, parts_ref, b_ref, g_ref, be_ref, rm_ref,
              rv_ref, w_ref, out_ref):
    dis = _dis_block(parts_ref)
    h = (_psum(agg_ref) + y_ref[...].astype(jnp.float32)) * dis + b_ref[...]
    h = jnp.maximum(h, 0.0)
    scale = g_ref[...] * lax.rsqrt(rv_ref[...] + BN_EPS)
    h = (h - rm_ref[...]) * scale + be_ref[...]
    out_ref[...] = jnp.dot(h * dis, w_ref[...],
                           preferred_element_type=jnp.float32).astype(
                               jnp.bfloat16)


def _tc3_body(agg_ref, y_ref, parts_ref, b_ref, g_ref, be_ref, rm_ref,
              rv_ref, out_ref):
    dis = _dis_block(parts_ref)
    h = (_psum(agg_ref) + y_ref[...].astype(jnp.float32)) * dis + b_ref[...]
    scale = g_ref[...] * lax.rsqrt(rv_ref[...] + BN_EPS)
    out_ref[...] = (h - rm_ref[...]) * scale + be_ref[...]


def _row_spec(shape3=None, shape2=None):
    if shape3 is not None:
        return pl.BlockSpec(shape3, lambda i: (0, i, 0))
    return pl.BlockSpec(shape2, lambda i: (i, 0))


def _full_spec(shape):
    nd = len(shape)
    return pl.BlockSpec(shape, lambda i: (0,) * nd)


def _tc0(x, W1, parts):
    return pl.pallas_call(
        _tc0_body,
        grid=(N // BM,),
        in_specs=[_row_spec(shape2=(BM, 128)),
                  _full_spec((128, 256)),
                  _row_spec(shape3=(2, BM, 16))],
        out_specs=_row_spec(shape3=(2, BM, 128)),
        out_shape=jax.ShapeDtypeStruct((2, N, 128), jnp.bfloat16),
    )(x, W1, parts)


def _tc1(agga, aggb, y, parts, b, g, be, rm, rv, W2):
    return pl.pallas_call(
        _tc1_body,
        grid=(N // BM,),
        in_specs=[_row_spec(shape3=(2, BM, 128)),
                  _row_spec(shape3=(2, BM, 128)),
                  _row_spec(shape3=(2, BM, 128)),
                  _row_spec(shape3=(2, BM, 16)),
                  _full_spec((1, 256)), _full_spec((1, 256)),
                  _full_spec((1, 256)), _full_spec((1, 256)),
                  _full_spec((1, 256)),
                  _full_spec((256, 128))],
        out_specs=_row_spec(shape2=(BM, 128)),
        out_shape=jax.ShapeDtypeStruct((N, 128), jnp.bfloat16),
    )(agga, aggb, y, parts, b, g, be, rm, rv, W2)


def _tc2(agg, y, parts, b, g, be, rm, rv, W3):
    return pl.pallas_call(
        _tc2_body,
        grid=(N // BM,),
        in_specs=[_row_spec(shape3=(2, BM, 128)),
                  _row_spec(shape2=(BM, 128)),
                  _row_spec(shape3=(2, BM, 16)),
                  _full_spec((1, 128)), _full_spec((1, 128)),
                  _full_spec((1, 128)), _full_spec((1, 128)),
                  _full_spec((1, 128)),
                  _full_spec((128, 128))],
        out_specs=_row_spec(shape2=(BM, 128)),
        out_shape=jax.ShapeDtypeStruct((N, 128), jnp.bfloat16),
    )(agg, y, parts, b, g, be, rm, rv, W3)


def _tc3(agg, y, parts, b, g, be, rm, rv):
    return pl.pallas_call(
        _tc3_body,
        grid=(N // BM,),
        in_specs=[_row_spec(shape3=(2, BM, 128)),
                  _row_spec(shape2=(BM, 128)),
                  _row_spec(shape3=(2, BM, 16)),
                  _full_spec((1, 128)), _full_spec((1, 128)),
                  _full_spec((1, 128)), _full_spec((1, 128)),
                  _full_spec((1, 128))],
        out_specs=_row_spec(shape2=(BM, 128)),
        out_shape=jax.ShapeDtypeStruct((N, 128), jnp.float32),
    )(agg, y, parts, b, g, be, rm, rv)


def kernel(x, edge_index, W1, b1, g1, be1, rm1, rv1,
           W2, b2, g2, be2, rm2, rv2, W3, b3, g3, be3, rm3, rv3):
    ei = edge_index.astype(jnp.int32)
    pad = E_PAD - E
    src = jnp.concatenate(
        [ei[0], jnp.zeros((pad,), jnp.int32)]).reshape(NBLK, BE)
    dst = jnp.concatenate(
        [ei[1], N + (jnp.arange(pad, dtype=jnp.int32) % 16)]).reshape(NBLK, BE)
    zeros128 = jnp.zeros((NS, 128), jnp.bfloat16)
    zeros16 = jnp.zeros((NS, 16), jnp.float32)
    ones = jnp.ones((BE, 16), jnp.float32)
    r = lambda v: v.reshape(1, -1)

    parts = _deg_kernel(dst, ones, zeros16)
    y1 = _tc0(x, W1, parts)
    agg1a = _agg(y1[0], src, dst, zeros128)
    agg1b = _agg(y1[1], src, dst, zeros128)
    y2 = _tc1(agg1a, agg1b, y1, parts,
              r(b1), r(g1), r(be1), r(rm1), r(rv1), W2)
    agg2 = _agg(y2, src, dst, zeros128)
    y3 = _tc2(agg2, y2, parts, r(b2), r(g2), r(be2), r(rm2), r(rv2), W3)
    agg3 = _agg(y3, src, dst, zeros128)
    return _tc3(agg3, y3, parts, r(b3), r(g3), r(be3), r(rm3), r(rv3))


# speedup vs baseline: 1.4293x; 1.4293x over previous
import functools

import jax
import jax.numpy as jnp
from jax import lax
from jax.experimental import pallas as pl
from jax.experimental.pallas import tpu as pltpu
from jax.experimental.pallas import tpu_sc as plsc

N = 10000
NS = N + 16
E = 320000
BE = 128
NBLK = 2560
E_PAD = NBLK * BE
BN_EPS = 1e-5
BM = 1000

_MESH = plsc.VectorSubcoreMesh(core_axis_name="c", subcore_axis_name="s")
_SC_PARAMS = pltpu.CompilerParams(use_tc_tiling_on_sc=False)


@functools.partial(
    pl.kernel,
    out_type=jax.ShapeDtypeStruct((2, N, 16), jnp.float32),
    mesh=_MESH,
    scratch_types=[
        pltpu.VMEM((NBLK // 32, BE), jnp.int32),
        pltpu.VMEM((BE, 16), jnp.float32),
        pltpu.VMEM((125, 16), jnp.float32),
        pltpu.VMEM_SHARED((NS, 16), jnp.float32),
    ],
    compiler_params=_SC_PARAMS,
)
def _deg_kernel(dst_hbm, ones_hbm, zeros_hbm, out_hbm,
                dst_v, ones_v, obuf, acc_sh):
    cid = lax.axis_index("c")
    sid = lax.axis_index("s")
    nbw = NBLK // 32
    pltpu.sync_copy(zeros_hbm.at[pl.ds(sid * 626, 626)],
                    acc_sh.at[pl.ds(sid * 626, 626)])
    pltpu.sync_copy(ones_hbm, ones_v)
    base = cid * (NBLK // 2) + sid * nbw
    pltpu.sync_copy(dst_hbm.at[pl.ds(base, nbw)], dst_v)
    plsc.subcore_barrier()

    def body(b, carry):
        pltpu.sync_copy(ones_v, acc_sh.at[dst_v.at[b]], add=True)
        return carry

    lax.fori_loop(0, nbw, body, 0)
    plsc.subcore_barrier()
    for j in range(5):
        r = sid * 625 + j * 125
        pltpu.sync_copy(acc_sh.at[pl.ds(r, 125)], obuf)
        pltpu.sync_copy(obuf, out_hbm.at[cid, pl.ds(r, 125)])


_NBW = NBLK // 16
_K = 8


@functools.partial(
    pl.kernel,
    out_type=jax.ShapeDtypeStruct((2, N, 64), jnp.bfloat16),
    mesh=_MESH,
    scratch_types=[
        pltpu.VMEM((_NBW, BE), jnp.int32),
        pltpu.VMEM((_NBW, BE), jnp.int32),
        [pltpu.VMEM((BE, 64), jnp.bfloat16)] * _K,
        pltpu.VMEM((125, 64), jnp.bfloat16),
        pltpu.VMEM_SHARED((NS, 64), jnp.bfloat16),
        [pltpu.SemaphoreType.DMA] * _K,
    ],
    compiler_params=_SC_PARAMS,
)
def _agg(y0_hbm, y1_hbm, src_hbm, dst_hbm, zeros_hbm, out_hbm,
         src_v, dst_v, rows, obuf, acc_sh, sems):
    cid = lax.axis_index("c")
    sid = lax.axis_index("s")
    pltpu.sync_copy(zeros_hbm.at[pl.ds(sid * 626, 626)],
                    acc_sh.at[pl.ds(sid * 626, 626)])
    base = sid * _NBW
    pltpu.sync_copy(src_hbm.at[pl.ds(base, _NBW)], src_v)
    pltpu.sync_copy(dst_hbm.at[pl.ds(base, _NBW)], dst_v)
    plsc.subcore_barrier()

    def run(y_ref):
        def body(g, carry):
            b = g * _K
            for k in range(_K):
                pltpu.async_copy(y_ref.at[src_v.at[b + k]], rows[k], sems[k])
            for k in range(_K):
                pltpu.make_async_copy(y_ref.at[src_v.at[b + k]], rows[k],
                                      sems[k]).wait()
                pltpu.sync_copy(rows[k], acc_sh.at[dst_v.at[b + k]], add=True)
            return carry
        lax.fori_loop(0, _NBW // _K, body, 0)

    pl.when(cid == 0)(lambda: run(y0_hbm))
    pl.when(cid == 1)(lambda: run(y1_hbm))
    plsc.subcore_barrier()
    for j in range(5):
        r = sid * 625 + j * 125
        pltpu.sync_copy(acc_sh.at[pl.ds(r, 125)], obuf)
        pltpu.sync_copy(obuf, out_hbm.at[cid, pl.ds(r, 125)])


def _dis_block(parts):
    deg = parts[0][:, 0:1] + parts[1][:, 0:1] + 1.0
    return lax.rsqrt(deg)


def _cat(ref):
    return jnp.concatenate(
        [ref[q] for q in range(ref.shape[0])], axis=1).astype(jnp.float32)


def _split_out(out_ref, y):
    q = y.shape[1] // out_ref.shape[0]
    y = y.astype(out_ref.dtype)
    for i in range(out_ref.shape[0]):
        out_ref[i] = y[:, i * q:(i + 1) * q]


def _tc0_body(x_ref, w_ref, parts_ref, out_ref):
    dis = _dis_block(parts_ref)
    y = jnp.dot(x_ref[...], w_ref[...],
                preferred_element_type=jnp.float32) * dis
    _split_out(out_ref, y)


def _tc1_body(agga_ref, aggb_ref, y_ref, parts_ref, b_ref, g_ref, be_ref,
              rm_ref, rv_ref, w_ref, out_ref):
    dis = _dis_block(parts_ref)
    agg = jnp.concatenate(
        [agga_ref[0], agga_ref[1], aggb_ref[0], aggb_ref[1]],
        axis=1).astype(jnp.float32)
    h = (agg + _cat(y_ref)) * dis + b_ref[...]
    h = jnp.maximum(h, 0.0)
    scale = g_ref[...] * lax.rsqrt(rv_ref[...] + BN_EPS)
    h = (h - rm_ref[...]) * scale + be_ref[...]
    _split_out(out_ref, jnp.dot(h * dis, w_ref[...],
                                preferred_element_type=jnp.float32))


def _tc2_body(agg_ref, y_ref, parts_ref, b_ref, g_ref, be_ref, rm_ref,
              rv_ref, w_ref, out_ref):
    dis = _dis_block(parts_ref)
    h = (_cat(agg_ref) + _cat(y_ref)) * dis + b_ref[...]
    h = jnp.maximum(h, 0.0)
    scale = g_ref[...] * lax.rsqrt(rv_ref[...] + BN_EPS)
    h = (h - rm_ref[...]) * scale + be_ref[...]
    _split_out(out_ref, jnp.dot(h * dis, w_ref[...],
                                preferred_element_type=jnp.float32))


def _tc3_body(agg_ref, y_ref, parts_ref, b_ref, g_ref, be_ref, rm_ref,
              rv_ref, out_ref):
    dis = _dis_block(parts_ref)
    h = (_cat(agg_ref) + _cat(y_ref)) * dis + b_ref[...]
    scale = g_ref[...] * lax.rsqrt(rv_ref[...] + BN_EPS)
    out_ref[...] = (h - rm_ref[...]) * scale + be_ref[...]


def _row_spec(shape3=None, shape2=None):
    if shape3 is not None:
        return pl.BlockSpec(shape3, lambda i: (0, i, 0))
    return pl.BlockSpec(shape2, lambda i: (i, 0))


def _full_spec(shape):
    nd = len(shape)
    return pl.BlockSpec(shape, lambda i: (0,) * nd)


def _tc0(x, W1, parts):
    return pl.pallas_call(
        _tc0_body,
        grid=(N // BM,),
        in_specs=[_row_spec(shape2=(BM, 128)),
                  _full_spec((128, 256)),
                  _row_spec(shape3=(2, BM, 16))],
        out_specs=_row_spec(shape3=(4, BM, 64)),
        out_shape=jax.ShapeDtypeStruct((4, N, 64), jnp.bfloat16),
    )(x, W1, parts)


def _tc1(agga, aggb, y, parts, b, g, be, rm, rv, W2):
    return pl.pallas_call(
        _tc1_body,
        grid=(N // BM,),
        in_specs=[_row_spec(shape3=(2, BM, 64)),
                  _row_spec(shape3=(2, BM, 64)),
                  _row_spec(shape3=(4, BM, 64)),
                  _row_spec(shape3=(2, BM, 16)),
                  _full_spec((1, 256)), _full_spec((1, 256)),
                  _full_spec((1, 256)), _full_spec((1, 256)),
                  _full_spec((1, 256)),
                  _full_spec((256, 128))],
        out_specs=_row_spec(shape3=(2, BM, 64)),
        out_shape=jax.ShapeDtypeStruct((2, N, 64), jnp.bfloat16),
    )(agga, aggb, y, parts, b, g, be, rm, rv, W2)


def _tc2(agg, y, parts, b, g, be, rm, rv, W3):
    return pl.pallas_call(
        _tc2_body,
        grid=(N // BM,),
        in_specs=[_row_spec(shape3=(2, BM, 64)),
                  _row_spec(shape3=(2, BM, 64)),
                  _row_spec(shape3=(2, BM, 16)),
                  _full_spec((1, 128)), _full_spec((1, 128)),
                  _full_spec((1, 128)), _full_spec((1, 128)),
                  _full_spec((1, 128)),
                  _full_spec((128, 128))],
        out_specs=_row_spec(shape3=(2, BM, 64)),
        out_shape=jax.ShapeDtypeStruct((2, N, 64), jnp.bfloat16),
    )(agg, y, parts, b, g, be, rm, rv, W3)


def _tc3(agg, y, parts, b, g, be, rm, rv):
    return pl.pallas_call(
        _tc3_body,
        grid=(N // BM,),
        in_specs=[_row_spec(shape3=(2, BM, 64)),
                  _row_spec(shape3=(2, BM, 64)),
                  _row_spec(shape3=(2, BM, 16)),
                  _full_spec((1, 128)), _full_spec((1, 128)),
                  _full_spec((1, 128)), _full_spec((1, 128)),
                  _full_spec((1, 128))],
        out_specs=_row_spec(shape2=(BM, 128)),
        out_shape=jax.ShapeDtypeStruct((N, 128), jnp.float32),
    )(agg, y, parts, b, g, be, rm, rv)


def kernel(x, edge_index, W1, b1, g1, be1, rm1, rv1,
           W2, b2, g2, be2, rm2, rv2, W3, b3, g3, be3, rm3, rv3):
    ei = edge_index.astype(jnp.int32)
    pad = E_PAD - E
    src = jnp.concatenate(
        [ei[0], jnp.zeros((pad,), jnp.int32)]).reshape(NBLK, BE)
    dst = jnp.concatenate(
        [ei[1], N + (jnp.arange(pad, dtype=jnp.int32) % 16)]).reshape(NBLK, BE)
    zeros64 = jnp.zeros((NS, 64), jnp.bfloat16)
    zeros16 = jnp.zeros((NS, 16), jnp.float32)
    ones = jnp.ones((BE, 16), jnp.float32)
    r = lambda v: v.reshape(1, -1)

    parts = _deg_kernel(dst, ones, zeros16)
    y1 = _tc0(x, W1, parts)
    agg1a = _agg(y1[0], y1[1], src, dst, zeros64)
    agg1b = _agg(y1[2], y1[3], src, dst, zeros64)
    y2 = _tc1(agg1a, agg1b, y1, parts,
              r(b1), r(g1), r(be1), r(rm1), r(rv1), W2)
    agg2 = _agg(y2[0], y2[1], src, dst, zeros64)
    y3 = _tc2(agg2, y2, parts, r(b2), r(g2), r(be2), r(rm2), r(rv2), W3)
    agg3 = _agg(y3[0], y3[1], src, dst, zeros64)
    return _tc3(agg3, y3, parts, r(b3), r(g3), r(be3), r(rm3), r(rv3))


# speedup vs baseline: 1.5804x; 1.1057x over previous
import functools

import jax
import jax.numpy as jnp
from jax import lax
from jax.experimental import pallas as pl
from jax.experimental.pallas import tpu as pltpu
from jax.experimental.pallas import tpu_sc as plsc

N = 10000
NS = N + 16
E = 320000
BE = 128
NBLK = 2560
E_PAD = NBLK * BE
BN_EPS = 1e-5
BM = 1000

_MESH = plsc.VectorSubcoreMesh(core_axis_name="c", subcore_axis_name="s")
_SC_PARAMS = pltpu.CompilerParams(use_tc_tiling_on_sc=False)


@functools.partial(
    pl.kernel,
    out_type=jax.ShapeDtypeStruct((2, N, 16), jnp.float32),
    mesh=_MESH,
    scratch_types=[
        pltpu.VMEM((NBLK // 32, BE), jnp.int32),
        pltpu.VMEM((BE, 16), jnp.float32),
        pltpu.VMEM((125, 16), jnp.float32),
        pltpu.VMEM_SHARED((NS, 16), jnp.float32),
    ],
    compiler_params=_SC_PARAMS,
)
def _deg_kernel(dst_hbm, ones_hbm, zeros_hbm, out_hbm,
                dst_v, ones_v, obuf, acc_sh):
    cid = lax.axis_index("c")
    sid = lax.axis_index("s")
    nbw = NBLK // 32
    pltpu.sync_copy(zeros_hbm.at[pl.ds(sid * 626, 626)],
                    acc_sh.at[pl.ds(sid * 626, 626)])
    pltpu.sync_copy(ones_hbm, ones_v)
    base = cid * (NBLK // 2) + sid * nbw
    pltpu.sync_copy(dst_hbm.at[pl.ds(base, nbw)], dst_v)
    plsc.subcore_barrier()

    def body(b, carry):
        pltpu.sync_copy(ones_v, acc_sh.at[dst_v.at[b]], add=True)
        return carry

    lax.fori_loop(0, nbw, body, 0)
    plsc.subcore_barrier()
    for j in range(5):
        r = sid * 625 + j * 125
        pltpu.sync_copy(acc_sh.at[pl.ds(r, 125)], obuf)
        pltpu.sync_copy(obuf, out_hbm.at[cid, pl.ds(r, 125)])


_NBW = NBLK // 16
_K = 4
_G = 2 * _K


@functools.partial(
    pl.kernel,
    out_type=jax.ShapeDtypeStruct((2, N, 64), jnp.bfloat16),
    mesh=_MESH,
    scratch_types=[
        pltpu.VMEM((_NBW, BE), jnp.int32),
        pltpu.VMEM((_NBW, BE), jnp.int32),
        [pltpu.VMEM((BE, 64), jnp.bfloat16)] * _G,
        pltpu.VMEM((125, 64), jnp.bfloat16),
        pltpu.VMEM_SHARED((NS, 64), jnp.bfloat16),
        [pltpu.SemaphoreType.DMA] * _G,
        [pltpu.SemaphoreType.DMA] * _G,
    ],
    compiler_params=_SC_PARAMS,
)
def _agg(y0_hbm, y1_hbm, src_hbm, dst_hbm, zeros_hbm, out_hbm,
         src_v, dst_v, rows, obuf, acc_sh, gsems, ssems):
    cid = lax.axis_index("c")
    sid = lax.axis_index("s")
    pltpu.sync_copy(zeros_hbm.at[pl.ds(sid * 626, 626)],
                    acc_sh.at[pl.ds(sid * 626, 626)])
    base = sid * _NBW
    pltpu.sync_copy(src_hbm.at[pl.ds(base, _NBW)], src_v)
    pltpu.sync_copy(dst_hbm.at[pl.ds(base, _NBW)], dst_v)
    plsc.subcore_barrier()

    def run(y_ref):
        def gather(blk, buf):
            pltpu.async_copy(y_ref.at[src_v.at[blk]], rows[buf], gsems[buf])

        def gwait(blk, buf):
            pltpu.make_async_copy(y_ref.at[src_v.at[blk]], rows[buf],
                                  gsems[buf]).wait()

        def scatter(blk, buf):
            pltpu.async_copy(rows[buf], acc_sh.at[dst_v.at[blk]], ssems[buf],
                             add=True)

        def swait(blk, buf):
            pltpu.make_async_copy(rows[buf], acc_sh.at[dst_v.at[blk]],
                                  ssems[buf]).wait()

        for k in range(_K):
            gather(k, k)

        def body(g, carry):
            b = g * _G
            for k in range(_K):
                gather(b + _K + k, _K + k)
            for k in range(_K):
                gwait(b + k, k)
                scatter(b + k, k)
            for k in range(_K):
                swait(b + k, k)

            @pl.when(b + _G < _NBW)
            def _():
                for k in range(_K):
                    gather(b + _G + k, k)

            for k in range(_K):
                gwait(b + _K + k, _K + k)
                scatter(b + _K + k, _K + k)
            for k in range(_K):
                swait(b + _K + k, _K + k)
            return carry

        lax.fori_loop(0, _NBW // _G, body, 0)

    pl.when(cid == 0)(lambda: run(y0_hbm))
    pl.when(cid == 1)(lambda: run(y1_hbm))
    plsc.subcore_barrier()
    for j in range(5):
        r = sid * 625 + j * 125
        pltpu.sync_copy(acc_sh.at[pl.ds(r, 125)], obuf)
        pltpu.sync_copy(obuf, out_hbm.at[cid, pl.ds(r, 125)])


def _dis_block(parts):
    deg = parts[0][:, 0:1] + parts[1][:, 0:1] + 1.0
    return lax.rsqrt(deg)


def _cat(ref):
    return jnp.concatenate(
        [ref[q] for q in range(ref.shape[0])], axis=1).astype(jnp.float32)


def _split_out(out_ref, y):
    q = y.shape[1] // out_ref.shape[0]
    y = y.astype(out_ref.dtype)
    for i in range(out_ref.shape[0]):
        out_ref[i] = y[:, i * q:(i + 1) * q]


def _tc0_body(x_ref, w_ref, parts_ref, out_ref):
    dis = _dis_block(parts_ref)
    y = jnp.dot(x_ref[...], w_ref[...],
                preferred_element_type=jnp.float32) * dis
    _split_out(out_ref, y)


def _tc1_body(agga_ref, aggb_ref, y_ref, parts_ref, b_ref, g_ref, be_ref,
              rm_ref, rv_ref, w_ref, out_ref):
    dis = _dis_block(parts_ref)
    agg = jnp.concatenate(
        [agga_ref[0], agga_ref[1], aggb_ref[0], aggb_ref[1]],
        axis=1).astype(jnp.float32)
    h = (agg + _cat(y_ref)) * dis + b_ref[...]
    h = jnp.maximum(h, 0.0)
    scale = g_ref[...] * lax.rsqrt(rv_ref[...] + BN_EPS)
    h = (h - rm_ref[...]) * scale + be_ref[...]
    _split_out(out_ref, jnp.dot(h * dis, w_ref[...],
                                preferred_element_type=jnp.float32))


def _tc2_body(agg_ref, y_ref, parts_ref, b_ref, g_ref, be_ref, rm_ref,
              rv_ref, w_ref, out_ref):
    dis = _dis_block(parts_ref)
    h = (_cat(agg_ref) + _cat(y_ref)) * dis + b_ref[...]
    h = jnp.maximum(h, 0.0)
    scale = g_ref[...] * lax.rsqrt(rv_ref[...] + BN_EPS)
    h = (h - rm_ref[...]) * scale + be_ref[...]
    _split_out(out_ref, jnp.dot(h * dis, w_ref[...],
                                preferred_element_type=jnp.float32))


def _tc3_body(agg_ref, y_ref, parts_ref, b_ref, g_ref, be_ref, rm_ref,
              rv_ref, out_ref):
    dis = _dis_block(parts_ref)
    h = (_cat(agg_ref) + _cat(y_ref)) * dis + b_ref[...]
    scale = g_ref[...] * lax.rsqrt(rv_ref[...] + BN_EPS)
    out_ref[...] = (h - rm_ref[...]) * scale + be_ref[...]


def _row_spec(shape3=None, shape2=None):
    if shape3 is not None:
        return pl.BlockSpec(shape3, lambda i: (0, i, 0))
    return pl.BlockSpec(shape2, lambda i: (i, 0))


def _full_spec(shape):
    nd = len(shape)
    return pl.BlockSpec(shape, lambda i: (0,) * nd)


def _tc0(x, W1, parts):
    return pl.pallas_call(
        _tc0_body,
        grid=(N // BM,),
        in_specs=[_row_spec(shape2=(BM, 128)),
                  _full_spec((128, 256)),
                  _row_spec(shape3=(2, BM, 16))],
        out_specs=_row_spec(shape3=(4, BM, 64)),
        out_shape=jax.ShapeDtypeStruct((4, N, 64), jnp.bfloat16),
    )(x, W1, parts)


def _tc1(agga, aggb, y, parts, b, g, be, rm, rv, W2):
    return pl.pallas_call(
        _tc1_body,
        grid=(N // BM,),
        in_specs=[_row_spec(shape3=(2, BM, 64)),
                  _row_spec(shape3=(2, BM, 64)),
                  _row_spec(shape3=(4, BM, 64)),
                  _row_spec(shape3=(2, BM, 16)),
                  _full_spec((1, 256)), _full_spec((1, 256)),
                  _full_spec((1, 256)), _full_spec((1, 256)),
                  _full_spec((1, 256)),
                  _full_spec((256, 128))],
        out_specs=_row_spec(shape3=(2, BM, 64)),
        out_shape=jax.ShapeDtypeStruct((2, N, 64), jnp.bfloat16),
    )(agga, aggb, y, parts, b, g, be, rm, rv, W2)


def _tc2(agg, y, parts, b, g, be, rm, rv, W3):
    return pl.pallas_call(
        _tc2_body,
        grid=(N // BM,),
        in_specs=[_row_spec(shape3=(2, BM, 64)),
                  _row_spec(shape3=(2, BM, 64)),
                  _row_spec(shape3=(2, BM, 16)),
                  _full_spec((1, 128)), _full_spec((1, 128)),
                  _full_spec((1, 128)), _full_spec((1, 128)),
                  _full_spec((1, 128)),
                  _full_spec((128, 128))],
        out_specs=_row_spec(shape3=(2, BM, 64)),
        out_shape=jax.ShapeDtypeStruct((2, N, 64), jnp.bfloat16),
    )(agg, y, parts, b, g, be, rm, rv, W3)


def _tc3(agg, y, parts, b, g, be, rm, rv):
    return pl.pallas_call(
        _tc3_body,
        grid=(N // BM,),
        in_specs=[_row_spec(shape3=(2, BM, 64)),
                  _row_spec(shape3=(2, BM, 64)),
                  _row_spec(shape3=(2, BM, 16)),
                  _full_spec((1, 128)), _full_spec((1, 128)),
                  _full_spec((1, 128)), _full_spec((1, 128)),
                  _full_spec((1, 128))],
        out_specs=_row_spec(shape2=(BM, 128)),
        out_shape=jax.ShapeDtypeStruct((N, 128), jnp.float32),
    )(agg, y, parts, b, g, be, rm, rv)


def kernel(x, edge_index, W1, b1, g1, be1, rm1, rv1,
           W2, b2, g2, be2, rm2, rv2, W3, b3, g3, be3, rm3, rv3):
    ei = edge_index.astype(jnp.int32)
    pad = E_PAD - E
    src = jnp.concatenate(
        [ei[0], jnp.zeros((pad,), jnp.int32)]).reshape(NBLK, BE)
    dst = jnp.concatenate(
        [ei[1], N + (jnp.arange(pad, dtype=jnp.int32) % 16)]).reshape(NBLK, BE)
    zeros64 = jnp.zeros((NS, 64), jnp.bfloat16)
    zeros16 = jnp.zeros((NS, 16), jnp.float32)
    ones = jnp.ones((BE, 16), jnp.float32)
    r = lambda v: v.reshape(1, -1)

    parts = _deg_kernel(dst, ones, zeros16)
    y1 = _tc0(x, W1, parts)
    agg1a = _agg(y1[0], y1[1], src, dst, zeros64)
    agg1b = _agg(y1[2], y1[3], src, dst, zeros64)
    y2 = _tc1(agg1a, agg1b, y1, parts,
              r(b1), r(g1), r(be1), r(rm1), r(rv1), W2)
    agg2 = _agg(y2[0], y2[1], src, dst, zeros64)
    y3 = _tc2(agg2, y2, parts, r(b2), r(g2), r(be2), r(rm2), r(rv2), W3)
    agg3 = _agg(y3[0], y3[1], src, dst, zeros64)
    return _tc3(agg3, y3, parts, r(b3), r(g3), r(be3), r(rm3), r(rv3))


# speedup vs baseline: 2.0648x; 1.3065x over previous
import functools

import jax
import jax.numpy as jnp
from jax import lax
from jax.experimental import pallas as pl
from jax.experimental.pallas import tpu as pltpu
from jax.experimental.pallas import tpu_sc as plsc

N = 10000
NS = N + 16
E = 320000
BE = 128
NBLK = 2560
E_PAD = NBLK * BE
BN_EPS = 1e-5
BM = 1000

_MESH = plsc.VectorSubcoreMesh(core_axis_name="c", subcore_axis_name="s")
_SC_PARAMS = pltpu.CompilerParams(use_tc_tiling_on_sc=False)


@functools.partial(
    pl.kernel,
    out_type=jax.ShapeDtypeStruct((2, N, 16), jnp.float32),
    mesh=_MESH,
    scratch_types=[
        pltpu.VMEM((NBLK // 32, BE), jnp.int32),
        pltpu.VMEM((BE, 16), jnp.float32),
        pltpu.VMEM((125, 16), jnp.float32),
        pltpu.VMEM_SHARED((NS, 16), jnp.float32),
    ],
    compiler_params=_SC_PARAMS,
)
def _deg_kernel(dst_hbm, ones_hbm, zeros_hbm, out_hbm,
                dst_v, ones_v, obuf, acc_sh):
    cid = lax.axis_index("c")
    sid = lax.axis_index("s")
    nbw = NBLK // 32
    pltpu.sync_copy(zeros_hbm.at[pl.ds(sid * 626, 626)],
                    acc_sh.at[pl.ds(sid * 626, 626)])
    pltpu.sync_copy(ones_hbm, ones_v)
    base = cid * (NBLK // 2) + sid * nbw
    pltpu.sync_copy(dst_hbm.at[pl.ds(base, nbw)], dst_v)
    plsc.subcore_barrier()

    def body(b, carry):
        pltpu.sync_copy(ones_v, acc_sh.at[dst_v.at[b]], add=True)
        return carry

    lax.fori_loop(0, nbw, body, 0)
    plsc.subcore_barrier()
    for j in range(5):
        r = sid * 625 + j * 125
        pltpu.sync_copy(acc_sh.at[pl.ds(r, 125)], obuf)
        pltpu.sync_copy(obuf, out_hbm.at[cid, pl.ds(r, 125)])


_NBW = NBLK // 16
_K = 4
_G = 2 * _K


@functools.partial(
    pl.kernel,
    out_type=jax.ShapeDtypeStruct((2, N, 64), jnp.bfloat16),
    mesh=_MESH,
    scratch_types=[
        pltpu.VMEM((_NBW, BE), jnp.int32),
        pltpu.VMEM((_NBW, BE), jnp.int32),
        [pltpu.VMEM((BE, 64), jnp.bfloat16)] * _G,
        pltpu.VMEM((125, 64), jnp.bfloat16),
        pltpu.VMEM_SHARED((NS, 64), jnp.bfloat16),
        [pltpu.SemaphoreType.DMA] * _G,
        [pltpu.SemaphoreType.DMA] * _G,
    ],
    compiler_params=_SC_PARAMS,
)
def _agg(y0_hbm, y1_hbm, src_hbm, dst_hbm, zeros_hbm, out_hbm,
         src_v, dst_v, rows, obuf, acc_sh, gsems, ssems):
    cid = lax.axis_index("c")
    sid = lax.axis_index("s")
    pltpu.sync_copy(zeros_hbm.at[pl.ds(sid * 626, 626)],
                    acc_sh.at[pl.ds(sid * 626, 626)])
    base = sid * _NBW
    pltpu.sync_copy(src_hbm.at[pl.ds(base, _NBW)], src_v)
    pltpu.sync_copy(dst_hbm.at[pl.ds(base, _NBW)], dst_v)
    plsc.subcore_barrier()

    def run(y_ref):
        def gather(blk, buf):
            pltpu.async_copy(y_ref.at[src_v.at[blk]], rows[buf], gsems[buf])

        def gwait(blk, buf):
            pltpu.make_async_copy(y_ref.at[src_v.at[blk]], rows[buf],
                                  gsems[buf]).wait()

        def scatter(blk, buf):
            pltpu.async_copy(rows[buf], acc_sh.at[dst_v.at[blk]], ssems[buf],
                             add=True)

        def swait(blk, buf):
            pltpu.make_async_copy(rows[buf], acc_sh.at[dst_v.at[blk]],
                                  ssems[buf]).wait()

        for k in range(_K):
            gather(k, k)

        def body(g, carry):
            b = g * _G
            for k in range(_K):
                gather(b + _K + k, _K + k)
            for k in range(_K):
                gwait(b + k, k)
                scatter(b + k, k)
            for k in range(_K):
                swait(b + k, k)

            @pl.when(b + _G < _NBW)
            def _():
                for k in range(_K):
                    gather(b + _G + k, k)

            for k in range(_K):
                gwait(b + _K + k, _K + k)
                scatter(b + _K + k, _K + k)
            for k in range(_K):
                swait(b + _K + k, _K + k)
            return carry

        lax.fori_loop(0, _NBW // _G, body, 0)

    pl.when(cid == 0)(lambda: run(y0_hbm))
    pl.when(cid == 1)(lambda: run(y1_hbm))
    plsc.subcore_barrier()
    for j in range(5):
        r = sid * 625 + j * 125
        pltpu.sync_copy(acc_sh.at[pl.ds(r, 125)], obuf)
        pltpu.sync_copy(obuf, out_hbm.at[cid, pl.ds(r, 125)])


def _dis_block(parts):
    deg = parts[0][:, 0:1] + parts[1][:, 0:1] + 1.0
    return lax.rsqrt(deg)


def _cat(ref):
    return jnp.concatenate(
        [ref[q] for q in range(ref.shape[0])], axis=1).astype(jnp.float32)


def _split_out(out_ref, y):
    q = y.shape[1] // out_ref.shape[0]
    y = y.astype(out_ref.dtype)
    for i in range(out_ref.shape[0]):
        out_ref[i] = y[:, i * q:(i + 1) * q]


def _tc0_body(x_ref, parts_ref, out_ref):
    dis = _dis_block(parts_ref)
    _split_out(out_ref, x_ref[...] * dis)


def _tc1_body(agg_ref, xp_ref, parts_ref, b_ref, g_ref, be_ref,
              rm_ref, rv_ref, w1_ref, w2_ref, out_ref):
    dis = _dis_block(parts_ref)
    hpre = (_cat(agg_ref) + _cat(xp_ref)) * dis
    h = jnp.dot(hpre, w1_ref[...],
                preferred_element_type=jnp.float32) + b_ref[...]
    h = jnp.maximum(h, 0.0)
    scale = g_ref[...] * lax.rsqrt(rv_ref[...] + BN_EPS)
    h = (h - rm_ref[...]) * scale + be_ref[...]
    _split_out(out_ref, jnp.dot(h * dis, w2_ref[...],
                                preferred_element_type=jnp.float32))


def _tc2_body(agg_ref, y_ref, parts_ref, b_ref, g_ref, be_ref, rm_ref,
              rv_ref, w_ref, out_ref):
    dis = _dis_block(parts_ref)
    h = (_cat(agg_ref) + _cat(y_ref)) * dis + b_ref[...]
    h = jnp.maximum(h, 0.0)
    scale = g_ref[...] * lax.rsqrt(rv_ref[...] + BN_EPS)
    h = (h - rm_ref[...]) * scale + be_ref[...]
    _split_out(out_ref, jnp.dot(h * dis, w_ref[...],
                                preferred_element_type=jnp.float32))


def _tc3_body(agg_ref, y_ref, parts_ref, b_ref, g_ref, be_ref, rm_ref,
              rv_ref, out_ref):
    dis = _dis_block(parts_ref)
    h = (_cat(agg_ref) + _cat(y_ref)) * dis + b_ref[...]
    scale = g_ref[...] * lax.rsqrt(rv_ref[...] + BN_EPS)
    out_ref[...] = (h - rm_ref[...]) * scale + be_ref[...]


def _row_spec(shape3=None, shape2=None):
    if shape3 is not None:
        return pl.BlockSpec(shape3, lambda i: (0, i, 0))
    return pl.BlockSpec(shape2, lambda i: (i, 0))


def _full_spec(shape):
    nd = len(shape)
    return pl.BlockSpec(shape, lambda i: (0,) * nd)


def _tc0(x, parts):
    return pl.pallas_call(
        _tc0_body,
        grid=(N // BM,),
        in_specs=[_row_spec(shape2=(BM, 128)),
                  _row_spec(shape3=(2, BM, 16))],
        out_specs=_row_spec(shape3=(2, BM, 64)),
        out_shape=jax.ShapeDtypeStruct((2, N, 64), jnp.bfloat16),
    )(x, parts)


def _tc1(agg, xp, parts, b, g, be, rm, rv, W1, W2):
    return pl.pallas_call(
        _tc1_body,
        grid=(N // BM,),
        in_specs=[_row_spec(shape3=(2, BM, 64)),
                  _row_spec(shape3=(2, BM, 64)),
                  _row_spec(shape3=(2, BM, 16)),
                  _full_spec((1, 256)), _full_spec((1, 256)),
                  _full_spec((1, 256)), _full_spec((1, 256)),
                  _full_spec((1, 256)),
                  _full_spec((128, 256)),
                  _full_spec((256, 128))],
        out_specs=_row_spec(shape3=(2, BM, 64)),
        out_shape=jax.ShapeDtypeStruct((2, N, 64), jnp.bfloat16),
    )(agg, xp, parts, b, g, be, rm, rv, W1, W2)


def _tc2(agg, y, parts, b, g, be, rm, rv, W3):
    return pl.pallas_call(
        _tc2_body,
        grid=(N // BM,),
        in_specs=[_row_spec(shape3=(2, BM, 64)),
                  _row_spec(shape3=(2, BM, 64)),
                  _row_spec(shape3=(2, BM, 16)),
                  _full_spec((1, 128)), _full_spec((1, 128)),
                  _full_spec((1, 128)), _full_spec((1, 128)),
                  _full_spec((1, 128)),
                  _full_spec((128, 128))],
        out_specs=_row_spec(shape3=(2, BM, 64)),
        out_shape=jax.ShapeDtypeStruct((2, N, 64), jnp.bfloat16),
    )(agg, y, parts, b, g, be, rm, rv, W3)


def _tc3(agg, y, parts, b, g, be, rm, rv):
    return pl.pallas_call(
        _tc3_body,
        grid=(N // BM,),
        in_specs=[_row_spec(shape3=(2, BM, 64)),
                  _row_spec(shape3=(2, BM, 64)),
                  _row_spec(shape3=(2, BM, 16)),
                  _full_spec((1, 128)), _full_spec((1, 128)),
                  _full_spec((1, 128)), _full_spec((1, 128)),
                  _full_spec((1, 128))],
        out_specs=_row_spec(shape2=(BM, 128)),
        out_shape=jax.ShapeDtypeStruct((N, 128), jnp.float32),
    )(agg, y, parts, b, g, be, rm, rv)


def kernel(x, edge_index, W1, b1, g1, be1, rm1, rv1,
           W2, b2, g2, be2, rm2, rv2, W3, b3, g3, be3, rm3, rv3):
    ei = edge_index.astype(jnp.int32)
    pad = E_PAD - E
    src = jnp.concatenate(
        [ei[0], jnp.zeros((pad,), jnp.int32)]).reshape(NBLK, BE)
    dst = jnp.concatenate(
        [ei[1], N + (jnp.arange(pad, dtype=jnp.int32) % 16)]).reshape(NBLK, BE)
    zeros64 = jnp.zeros((NS, 64), jnp.bfloat16)
    zeros16 = jnp.zeros((NS, 16), jnp.float32)
    ones = jnp.ones((BE, 16), jnp.float32)
    r = lambda v: v.reshape(1, -1)

    parts = _deg_kernel(dst, ones, zeros16)
    xp = _tc0(x, parts)
    agg1 = _agg(xp[0], xp[1], src, dst, zeros64)
    y2 = _tc1(agg1, xp, parts,
              r(b1), r(g1), r(be1), r(rm1), r(rv1), W1, W2)
    agg2 = _agg(y2[0], y2[1], src, dst, zeros64)
    y3 = _tc2(agg2, y2, parts, r(b2), r(g2), r(be2), r(rm2), r(rv2), W3)
    agg3 = _agg(y3[0], y3[1], src, dst, zeros64)
    return _tc3(agg3, y3, parts, r(b3), r(g3), r(be3), r(rm3), r(rv3))
